# initial kernel scaffold (unmeasured)
import jax
import jax.numpy as jnp
from jax import lax
from jax.experimental import pallas as pl
from jax.experimental.pallas import tpu as pltpu

N_DEV = 4
M_BLK = 2048
K_BLK = 2048
K = 8192
N = 4096


def _a2a(x):

    def body(x_ref, out_ref, bounce, local_sem, send_sems, recv_sems):
        i = lax.axis_index("i")

        barrier = pltpu.get_barrier_semaphore()
        for off in range(1, N_DEV):
            pl.semaphore_signal(
                barrier,
                inc=1,
                device_id=((i + off) % N_DEV,),
                device_id_type=pl.DeviceIdType.MESH,
            )
        pl.semaphore_wait(barrier, N_DEV - 1)

        sends = []
        for off in range(1, N_DEV):
            d = (i + off) % N_DEV
            rdma = pltpu.make_async_remote_copy(
                src_ref=x_ref.at[pl.ds(d * M_BLK, M_BLK), :],
                dst_ref=out_ref.at[:, pl.ds(i * K_BLK, K_BLK)],
                send_sem=send_sems.at[off],
                recv_sem=recv_sems.at[i],
                device_id=(d,),
                device_id_type=pl.DeviceIdType.MESH,
            )
            rdma.start()
            sends.append(rdma)

        c1 = pltpu.make_async_copy(
            x_ref.at[pl.ds(i * M_BLK, M_BLK), :], bounce, local_sem
        )
        c1.start()
        c1.wait()
        c2 = pltpu.make_async_copy(
            bounce, out_ref.at[:, pl.ds(i * K_BLK, K_BLK)], local_sem
        )
        c2.start()
        c2.wait()

        for off in range(1, N_DEV):
            src = (i + off) % N_DEV
            recv = pltpu.make_async_remote_copy(
                src_ref=x_ref.at[pl.ds(src * M_BLK, M_BLK), :],
                dst_ref=out_ref.at[:, pl.ds(src * K_BLK, K_BLK)],
                send_sem=send_sems.at[off],
                recv_sem=recv_sems.at[src],
                device_id=(src,),
                device_id_type=pl.DeviceIdType.MESH,
            )
            recv.wait_recv()
        for rdma in sends:
            rdma.wait_send()

    return pl.pallas_call(
        body,
        out_shape=jax.ShapeDtypeStruct((M_BLK, K), jnp.bfloat16),
        in_specs=[pl.BlockSpec(memory_space=pltpu.ANY)],
        out_specs=pl.BlockSpec(memory_space=pltpu.ANY),
        scratch_shapes=[
            pltpu.VMEM((M_BLK, K_BLK), jnp.bfloat16),
            pltpu.SemaphoreType.DMA,
            pltpu.SemaphoreType.DMA((N_DEV,)),
            pltpu.SemaphoreType.DMA((N_DEV,)),
        ],
        compiler_params=pltpu.CompilerParams(collective_id=0),
    )(x)


def _gemm(xr, w):
    bm, bn, bk = 1024, 2048, 2048

    def body(x_ref, w_ref, o_ref, acc_ref):
        @pl.when(pl.program_id(2) == 0)
        def _():
            acc_ref[...] = jnp.zeros_like(acc_ref)

        acc_ref[...] += jnp.dot(
            x_ref[...], w_ref[...], preferred_element_type=jnp.float32
        )

        @pl.when(pl.program_id(2) == pl.num_programs(2) - 1)
        def _():
            o_ref[...] = acc_ref[...]

    return pl.pallas_call(
        body,
        grid=(M_BLK // bm, N // bn, K // bk),
        in_specs=[
            pl.BlockSpec((bm, bk), lambda m, n, k: (m, k)),
            pl.BlockSpec((bk, bn), lambda m, n, k: (k, n)),
        ],
        out_specs=pl.BlockSpec((bm, bn), lambda m, n, k: (m, n)),
        out_shape=jax.ShapeDtypeStruct((M_BLK, N), jnp.float32),
        scratch_shapes=[pltpu.VMEM((bm, bn), jnp.float32)],
        compiler_params=pltpu.CompilerParams(
            dimension_semantics=("parallel", "parallel", "arbitrary"),
        ),
    )(xr, w)


def kernel(x, w_mat):
    xr = _a2a(x.astype(jnp.bfloat16))
    return _gemm(xr, w_mat.astype(jnp.bfloat16))


# baseline (device time: 441163 ns/iter reference)
import jax
import jax.numpy as jnp
from jax import lax
from jax.experimental import pallas as pl
from jax.experimental.pallas import tpu as pltpu

N_DEV = 4
M_BLK = 2048
K_BLK = 2048
K = 8192
N = 4096


def _a2a(x):

    def body(x_ref, out_ref, bounce, local_sem, send_sems, recv_sems):
        i = lax.axis_index("i")

        barrier = pltpu.get_barrier_semaphore()
        for off in range(1, N_DEV):
            pl.semaphore_signal(
                barrier,
                inc=1,
                device_id=((i + off) % N_DEV,),
                device_id_type=pl.DeviceIdType.MESH,
            )
        pl.semaphore_wait(barrier, N_DEV - 1)

        sends = []
        for off in range(1, N_DEV):
            d = (i + off) % N_DEV
            rdma = pltpu.make_async_remote_copy(
                src_ref=x_ref.at[pl.ds(d * M_BLK, M_BLK), :],
                dst_ref=out_ref.at[:, pl.ds(i * K_BLK, K_BLK)],
                send_sem=send_sems.at[off],
                recv_sem=recv_sems.at[i],
                device_id=(d,),
                device_id_type=pl.DeviceIdType.MESH,
            )
            rdma.start()
            sends.append(rdma)

        c1 = pltpu.make_async_copy(
            x_ref.at[pl.ds(i * M_BLK, M_BLK), :], bounce, local_sem
        )
        c1.start()
        c1.wait()
        c2 = pltpu.make_async_copy(
            bounce, out_ref.at[:, pl.ds(i * K_BLK, K_BLK)], local_sem
        )
        c2.start()
        c2.wait()

        for off in range(1, N_DEV):
            src = (i + off) % N_DEV
            recv = pltpu.make_async_remote_copy(
                src_ref=x_ref.at[pl.ds(src * M_BLK, M_BLK), :],
                dst_ref=out_ref.at[:, pl.ds(src * K_BLK, K_BLK)],
                send_sem=send_sems.at[off],
                recv_sem=recv_sems.at[src],
                device_id=(src,),
                device_id_type=pl.DeviceIdType.MESH,
            )
            recv.wait_recv()
        for rdma in sends:
            rdma.wait_send()

    return pl.pallas_call(
        body,
        out_shape=jax.ShapeDtypeStruct((M_BLK, K), jnp.bfloat16),
        in_specs=[pl.BlockSpec(memory_space=pl.ANY)],
        out_specs=pl.BlockSpec(memory_space=pl.ANY),
        scratch_shapes=[
            pltpu.MemorySpace.VMEM((M_BLK, K_BLK), jnp.bfloat16),
            pltpu.SemaphoreType.DMA,
            pltpu.SemaphoreType.DMA((N_DEV,)),
            pltpu.SemaphoreType.DMA((N_DEV,)),
        ],
        compiler_params=pltpu.CompilerParams(collective_id=0),
    )(x)


def _gemm(xr, w):
    bm, bn, bk = 1024, 2048, 2048

    def body(x_ref, w_ref, o_ref, acc_ref):
        @pl.when(pl.program_id(2) == 0)
        def _():
            acc_ref[...] = jnp.zeros_like(acc_ref)

        acc_ref[...] += jnp.dot(
            x_ref[...], w_ref[...], preferred_element_type=jnp.float32
        )

        @pl.when(pl.program_id(2) == pl.num_programs(2) - 1)
        def _():
            o_ref[...] = acc_ref[...]

    return pl.pallas_call(
        body,
        grid=(M_BLK // bm, N // bn, K // bk),
        in_specs=[
            pl.BlockSpec((bm, bk), lambda m, n, k: (m, k)),
            pl.BlockSpec((bk, bn), lambda m, n, k: (k, n)),
        ],
        out_specs=pl.BlockSpec((bm, bn), lambda m, n, k: (m, n)),
        out_shape=jax.ShapeDtypeStruct((M_BLK, N), jnp.float32),
        scratch_shapes=[pltpu.MemorySpace.VMEM((bm, bn), jnp.float32)],
        compiler_params=pltpu.CompilerParams(
            dimension_semantics=("parallel", "parallel", "arbitrary"),
            vmem_limit_bytes=60 * 1024 * 1024,
        ),
    )(xr, w)


def kernel(x, w_mat):
    xr = _a2a(x.astype(jnp.bfloat16))
    return _gemm(xr, w_mat.astype(jnp.bfloat16))


# device time: 371791 ns/iter; 1.1866x vs baseline; 1.1866x over previous
import jax
import jax.numpy as jnp
from jax import lax
from jax.experimental import pallas as pl
from jax.experimental.pallas import tpu as pltpu

N_DEV = 4
M_BLK = 2048
K_BLK = 2048
K = 8192
N = 4096
CAST_CHUNK = 256
NT = 512


def kernel(x, w_mat):
    def body(
        x_ref,
        w_ref,
        out_ref,
        xs_ref,
        recv_ref,
        cin,
        cout,
        b_buf,
        w_buf,
        cast_sem,
        b_sem,
        w_sem,
        send_sems,
        recv_sems,
    ):
        i = lax.axis_index("i")
        d1 = (i + 1) % N_DEV
        d2 = (i + 2) % N_DEV
        d3 = (i + 3) % N_DEV

        barrier = pltpu.get_barrier_semaphore()
        for off in range(1, N_DEV):
            pl.semaphore_signal(
                barrier,
                inc=1,
                device_id=((i + off) % N_DEV,),
                device_id_type=pl.DeviceIdType.MESH,
            )
        pl.semaphore_wait(barrier, N_DEV - 1)

        def cast_block(blk):

            def chunk(c, carry):
                rows = pl.ds(blk * M_BLK + c * CAST_CHUNK, CAST_CHUNK)
                cp = pltpu.make_async_copy(x_ref.at[rows, :], cin, cast_sem)
                cp.start()
                cp.wait()
                cout[...] = cin[...].astype(jnp.bfloat16)
                cp = pltpu.make_async_copy(cout, xs_ref.at[rows, :], cast_sem)
                cp.start()
                cp.wait()
                return carry

            lax.fori_loop(0, M_BLK // CAST_CHUNK, chunk, 0)

        def start_send(dst, sem_idx):
            rdma = pltpu.make_async_remote_copy(
                src_ref=xs_ref.at[pl.ds(dst * M_BLK, M_BLK), :],
                dst_ref=recv_ref.at[i],
                send_sem=send_sems.at[sem_idx],
                recv_sem=recv_sems.at[i],
                device_id=(dst,),
                device_id_type=pl.DeviceIdType.MESH,
            )
            rdma.start()
            return rdma

        def wait_recv_from(src):
            recv = pltpu.make_async_remote_copy(
                src_ref=recv_ref.at[src],
                dst_ref=recv_ref.at[src],
                send_sem=send_sems.at[0],
                recv_sem=recv_sems.at[src],
                device_id=(src,),
                device_id_type=pl.DeviceIdType.MESH,
            )
            recv.wait_recv()

        def gemm_block(j, src_ref):
            cp = pltpu.make_async_copy(src_ref, b_buf, b_sem)
            cp.start()
            cp.wait()

            def tile(t, carry):
                cols = pl.ds(t * NT, NT)
                wd = pltpu.make_async_copy(
                    w_ref.at[pl.ds(j * K_BLK, K_BLK), cols], w_buf, w_sem
                )
                wd.start()
                wd.wait()
                wt = w_buf[...].astype(jnp.bfloat16)
                out_ref[:, cols] += jnp.dot(
                    b_buf[...], wt, preferred_element_type=jnp.float32
                )
                return carry

            lax.fori_loop(0, N // NT, tile, 0)

        cast_block(d1)
        send1 = start_send(d1, 0)
        cast_block(d3)
        send3 = start_send(d3, 2)
        cast_block(d2)
        cast_block(i)

        out_ref[...] = jnp.zeros((M_BLK, N), jnp.float32)

        gemm_block(i, xs_ref.at[pl.ds(i * M_BLK, M_BLK), :])

        send1.wait_send()
        send3.wait_send()
        send2 = start_send(d2, 1)

        wait_recv_from(d3)
        gemm_block(d3, recv_ref.at[d3])
        wait_recv_from(d1)
        gemm_block(d1, recv_ref.at[d1])
        wait_recv_from(d2)
        gemm_block(d2, recv_ref.at[d2])
        send2.wait_send()

    out, _, _ = pl.pallas_call(
        body,
        out_shape=(
            jax.ShapeDtypeStruct((M_BLK, N), jnp.float32),
            jax.ShapeDtypeStruct((N_DEV * M_BLK, K_BLK), jnp.bfloat16),
            jax.ShapeDtypeStruct((N_DEV, M_BLK, K_BLK), jnp.bfloat16),
        ),
        in_specs=[
            pl.BlockSpec(memory_space=pl.ANY),
            pl.BlockSpec(memory_space=pl.ANY),
        ],
        out_specs=(
            pl.BlockSpec(memory_space=pltpu.MemorySpace.VMEM),
            pl.BlockSpec(memory_space=pl.ANY),
            pl.BlockSpec(memory_space=pl.ANY),
        ),
        scratch_shapes=[
            pltpu.MemorySpace.VMEM((CAST_CHUNK, K_BLK), jnp.float32),
            pltpu.MemorySpace.VMEM((CAST_CHUNK, K_BLK), jnp.bfloat16),
            pltpu.MemorySpace.VMEM((M_BLK, K_BLK), jnp.bfloat16),
            pltpu.MemorySpace.VMEM((K_BLK, NT), jnp.float32),
            pltpu.SemaphoreType.DMA,
            pltpu.SemaphoreType.DMA,
            pltpu.SemaphoreType.DMA,
            pltpu.SemaphoreType.DMA((N_DEV - 1,)),
            pltpu.SemaphoreType.DMA((N_DEV,)),
        ],
        compiler_params=pltpu.CompilerParams(
            collective_id=0,
            vmem_limit_bytes=60 * 1024 * 1024,
        ),
    )(x, w_mat)
    return out


# device time: 285315 ns/iter; 1.5462x vs baseline; 1.3031x over previous
import jax
import jax.numpy as jnp
from jax import lax
from jax.experimental import pallas as pl
from jax.experimental.pallas import tpu as pltpu

N_DEV = 4
M_BLK = 2048
K_BLK = 2048
K = 8192
N = 4096
CAST_CHUNK = 256
NT = 512


def kernel(x, w_mat):
    def body(
        x_ref,
        w_ref,
        out_ref,
        xs_ref,
        recv_ref,
        cin,
        cout,
        b_buf,
        w_buf,
        cin_sems,
        cout_sems,
        b_sem,
        w_sems,
        send_sems,
        recv_sems,
    ):
        i = lax.axis_index("i")
        d1 = (i + 1) % N_DEV
        d2 = (i + 2) % N_DEV
        d3 = (i + 3) % N_DEV

        barrier = pltpu.get_barrier_semaphore()
        for off in range(1, N_DEV):
            pl.semaphore_signal(
                barrier,
                inc=1,
                device_id=((i + off) % N_DEV,),
                device_id_type=pl.DeviceIdType.MESH,
            )
        pl.semaphore_wait(barrier, N_DEV - 1)

        def cast_block(blk):
            n_ch = M_BLK // CAST_CHUNK

            def in_copy(c, slot):
                rows = pl.ds(blk * M_BLK + c * CAST_CHUNK, CAST_CHUNK)
                return pltpu.make_async_copy(
                    x_ref.at[rows, :], cin.at[slot], cin_sems.at[slot]
                )

            def out_copy(c, slot):
                rows = pl.ds(blk * M_BLK + c * CAST_CHUNK, CAST_CHUNK)
                return pltpu.make_async_copy(
                    cout.at[slot], xs_ref.at[rows, :], cout_sems.at[slot]
                )

            in_copy(0, 0).start()

            def chunk(c, carry):
                slot = lax.rem(c, 2)

                @pl.when(c + 1 < n_ch)
                def _():
                    in_copy(c + 1, 1 - slot).start()

                in_copy(c, slot).wait()

                @pl.when(c >= 2)
                def _():
                    out_copy(c - 2, slot).wait()

                cout[slot] = cin[slot].astype(jnp.bfloat16)
                out_copy(c, slot).start()
                return carry

            lax.fori_loop(0, n_ch, chunk, 0)
            out_copy(n_ch - 2, (n_ch - 2) % 2).wait()
            out_copy(n_ch - 1, (n_ch - 1) % 2).wait()

        def start_send(dst, sem_idx):
            rdma = pltpu.make_async_remote_copy(
                src_ref=xs_ref.at[pl.ds(dst * M_BLK, M_BLK), :],
                dst_ref=recv_ref.at[i],
                send_sem=send_sems.at[sem_idx],
                recv_sem=recv_sems.at[i],
                device_id=(dst,),
                device_id_type=pl.DeviceIdType.MESH,
            )
            rdma.start()
            return rdma

        def wait_recv_from(src):
            recv = pltpu.make_async_remote_copy(
                src_ref=recv_ref.at[src],
                dst_ref=recv_ref.at[src],
                send_sem=send_sems.at[0],
                recv_sem=recv_sems.at[src],
                device_id=(src,),
                device_id_type=pl.DeviceIdType.MESH,
            )
            recv.wait_recv()

        def gemm_block(j, src_ref):
            n_t = N // NT
            cp = pltpu.make_async_copy(src_ref, b_buf, b_sem)
            cp.start()

            def w_copy(t, slot):
                return pltpu.make_async_copy(
                    w_ref.at[pl.ds(j * K_BLK, K_BLK), pl.ds(t * NT, NT)],
                    w_buf.at[slot],
                    w_sems.at[slot],
                )

            w_copy(0, 0).start()
            cp.wait()

            def tile(t, carry):
                slot = lax.rem(t, 2)

                @pl.when(t + 1 < n_t)
                def _():
                    w_copy(t + 1, 1 - slot).start()

                w_copy(t, slot).wait()
                wt = w_buf[slot].astype(jnp.bfloat16)
                cols = pl.ds(t * NT, NT)
                out_ref[:, cols] += jnp.dot(
                    b_buf[...], wt, preferred_element_type=jnp.float32
                )
                return carry

            lax.fori_loop(0, n_t, tile, 0)

        cast_block(d1)
        send1 = start_send(d1, 0)
        cast_block(d3)
        send3 = start_send(d3, 2)
        cast_block(d2)
        cast_block(i)

        out_ref[...] = jnp.zeros((M_BLK, N), jnp.float32)

        gemm_block(i, xs_ref.at[pl.ds(i * M_BLK, M_BLK), :])

        send1.wait_send()
        send3.wait_send()
        send2 = start_send(d2, 1)

        wait_recv_from(d3)
        gemm_block(d3, recv_ref.at[d3])
        wait_recv_from(d1)
        gemm_block(d1, recv_ref.at[d1])
        wait_recv_from(d2)
        gemm_block(d2, recv_ref.at[d2])
        send2.wait_send()

    out, _, _ = pl.pallas_call(
        body,
        out_shape=(
            jax.ShapeDtypeStruct((M_BLK, N), jnp.float32),
            jax.ShapeDtypeStruct((N_DEV * M_BLK, K_BLK), jnp.bfloat16),
            jax.ShapeDtypeStruct((N_DEV, M_BLK, K_BLK), jnp.bfloat16),
        ),
        in_specs=[
            pl.BlockSpec(memory_space=pl.ANY),
            pl.BlockSpec(memory_space=pl.ANY),
        ],
        out_specs=(
            pl.BlockSpec(memory_space=pltpu.MemorySpace.VMEM),
            pl.BlockSpec(memory_space=pl.ANY),
            pl.BlockSpec(memory_space=pl.ANY),
        ),
        scratch_shapes=[
            pltpu.MemorySpace.VMEM((2, CAST_CHUNK, K_BLK), jnp.float32),
            pltpu.MemorySpace.VMEM((2, CAST_CHUNK, K_BLK), jnp.bfloat16),
            pltpu.MemorySpace.VMEM((M_BLK, K_BLK), jnp.bfloat16),
            pltpu.MemorySpace.VMEM((2, K_BLK, NT), jnp.float32),
            pltpu.SemaphoreType.DMA((2,)),
            pltpu.SemaphoreType.DMA((2,)),
            pltpu.SemaphoreType.DMA,
            pltpu.SemaphoreType.DMA((2,)),
            pltpu.SemaphoreType.DMA((N_DEV - 1,)),
            pltpu.SemaphoreType.DMA((N_DEV,)),
        ],
        compiler_params=pltpu.CompilerParams(
            collective_id=0,
            vmem_limit_bytes=62 * 1024 * 1024,
        ),
    )(x, w_mat)
    return out


# device time: 281696 ns/iter; 1.5661x vs baseline; 1.0128x over previous
import jax
import jax.numpy as jnp
from jax import lax
from jax.experimental import pallas as pl
from jax.experimental.pallas import tpu as pltpu

N_DEV = 4
M_BLK = 2048
K_BLK = 2048
K = 8192
N = 4096
CAST_CHUNK = 256
NT = 512


def kernel(x, w_mat):
    def body(
        x_ref,
        w_ref,
        out_ref,
        xs_ref,
        recv_ref,
        cin,
        cout,
        b_buf,
        w_buf,
        cin_sems,
        cout_sems,
        b_sem,
        w_sems,
        send_sems,
        recv_sems,
    ):
        i = lax.axis_index("i")
        d1 = (i + 1) % N_DEV
        d2 = (i + 2) % N_DEV
        d3 = (i + 3) % N_DEV

        barrier = pltpu.get_barrier_semaphore()
        for off in range(1, N_DEV):
            pl.semaphore_signal(
                barrier,
                inc=1,
                device_id=((i + off) % N_DEV,),
                device_id_type=pl.DeviceIdType.MESH,
            )
        pl.semaphore_wait(barrier, N_DEV - 1)

        def cast_block(blk):
            n_ch = M_BLK // CAST_CHUNK

            def in_copy(c, slot):
                rows = pl.ds(blk * M_BLK + c * CAST_CHUNK, CAST_CHUNK)
                return pltpu.make_async_copy(
                    x_ref.at[rows, :], cin.at[slot], cin_sems.at[slot]
                )

            def out_copy(c, slot):
                rows = pl.ds(blk * M_BLK + c * CAST_CHUNK, CAST_CHUNK)
                return pltpu.make_async_copy(
                    cout.at[slot], xs_ref.at[rows, :], cout_sems.at[slot]
                )

            in_copy(0, 0).start()

            def chunk(c, carry):
                slot = lax.rem(c, 2)

                @pl.when(c + 1 < n_ch)
                def _():
                    in_copy(c + 1, 1 - slot).start()

                in_copy(c, slot).wait()

                @pl.when(c >= 2)
                def _():
                    out_copy(c - 2, slot).wait()

                cout[slot] = cin[slot].astype(jnp.bfloat16)
                out_copy(c, slot).start()
                return carry

            lax.fori_loop(0, n_ch, chunk, 0)
            out_copy(n_ch - 2, (n_ch - 2) % 2).wait()
            out_copy(n_ch - 1, (n_ch - 1) % 2).wait()

        def start_send(dst, sem_idx, r0=0, nr=M_BLK, rsem=None):
            rdma = pltpu.make_async_remote_copy(
                src_ref=xs_ref.at[pl.ds(dst * M_BLK + r0, nr), :],
                dst_ref=recv_ref.at[i, pl.ds(r0, nr), :],
                send_sem=send_sems.at[sem_idx],
                recv_sem=recv_sems.at[i if rsem is None else rsem],
                device_id=(dst,),
                device_id_type=pl.DeviceIdType.MESH,
            )
            rdma.start()
            return rdma

        def wait_recv_from(src, r0=0, nr=M_BLK, rsem=None):
            recv = pltpu.make_async_remote_copy(
                src_ref=recv_ref.at[src, pl.ds(r0, nr), :],
                dst_ref=recv_ref.at[src, pl.ds(r0, nr), :],
                send_sem=send_sems.at[0],
                recv_sem=recv_sems.at[src if rsem is None else rsem],
                device_id=(src,),
                device_id_type=pl.DeviceIdType.MESH,
            )
            recv.wait_recv()

        def gemm_block(j, src_ref, r0=0, nr=M_BLK):
            n_t = N // NT
            cp = pltpu.make_async_copy(src_ref, b_buf.at[pl.ds(0, nr), :], b_sem)
            cp.start()

            def w_copy(t, slot):
                return pltpu.make_async_copy(
                    w_ref.at[pl.ds(j * K_BLK, K_BLK), pl.ds(t * NT, NT)],
                    w_buf.at[slot],
                    w_sems.at[slot],
                )

            w_copy(0, 0).start()
            cp.wait()

            def tile(t, carry):
                slot = lax.rem(t, 2)

                @pl.when(t + 1 < n_t)
                def _():
                    w_copy(t + 1, 1 - slot).start()

                w_copy(t, slot).wait()
                wt = w_buf[slot].astype(jnp.bfloat16)
                cols = pl.ds(t * NT, NT)
                out_ref[pl.ds(r0, nr), cols] += jnp.dot(
                    b_buf[:nr], wt, preferred_element_type=jnp.float32
                )
                return carry

            lax.fori_loop(0, n_t, tile, 0)

        cast_block(d1)
        send1 = start_send(d1, 0)
        cast_block(d3)
        send3 = start_send(d3, 2)
        cast_block(d2)
        cast_block(i)

        out_ref[...] = jnp.zeros((M_BLK, N), jnp.float32)

        gemm_block(i, xs_ref.at[pl.ds(i * M_BLK, M_BLK), :])

        send1.wait_send()
        send3.wait_send()
        H = M_BLK // 2
        send2a = start_send(d2, 1, r0=0, nr=H)
        send2b = start_send(d2, 3, r0=H, nr=H, rsem=N_DEV)

        wait_recv_from(d3)
        gemm_block(d3, recv_ref.at[d3])
        wait_recv_from(d1)
        gemm_block(d1, recv_ref.at[d1])
        wait_recv_from(d2, r0=0, nr=H)
        gemm_block(d2, recv_ref.at[d2, pl.ds(0, H), :], r0=0, nr=H)
        wait_recv_from(d2, r0=H, nr=H, rsem=N_DEV)
        gemm_block(d2, recv_ref.at[d2, pl.ds(H, H), :], r0=H, nr=H)
        send2a.wait_send()
        send2b.wait_send()

    out, _, _ = pl.pallas_call(
        body,
        out_shape=(
            jax.ShapeDtypeStruct((M_BLK, N), jnp.float32),
            jax.ShapeDtypeStruct((N_DEV * M_BLK, K_BLK), jnp.bfloat16),
            jax.ShapeDtypeStruct((N_DEV, M_BLK, K_BLK), jnp.bfloat16),
        ),
        in_specs=[
            pl.BlockSpec(memory_space=pl.ANY),
            pl.BlockSpec(memory_space=pl.ANY),
        ],
        out_specs=(
            pl.BlockSpec(memory_space=pltpu.MemorySpace.VMEM),
            pl.BlockSpec(memory_space=pl.ANY),
            pl.BlockSpec(memory_space=pl.ANY),
        ),
        scratch_shapes=[
            pltpu.MemorySpace.VMEM((2, CAST_CHUNK, K_BLK), jnp.float32),
            pltpu.MemorySpace.VMEM((2, CAST_CHUNK, K_BLK), jnp.bfloat16),
            pltpu.MemorySpace.VMEM((M_BLK, K_BLK), jnp.bfloat16),
            pltpu.MemorySpace.VMEM((2, K_BLK, NT), jnp.float32),
            pltpu.SemaphoreType.DMA((2,)),
            pltpu.SemaphoreType.DMA((2,)),
            pltpu.SemaphoreType.DMA,
            pltpu.SemaphoreType.DMA((2,)),
            pltpu.SemaphoreType.DMA((N_DEV,)),
            pltpu.SemaphoreType.DMA((N_DEV + 1,)),
        ],
        compiler_params=pltpu.CompilerParams(
            collective_id=0,
            vmem_limit_bytes=62 * 1024 * 1024,
        ),
    )(x, w_mat)
    return out
